# baseline (speedup 1.0000x reference)
import functools

import jax
import jax.numpy as jnp
from jax import lax
from jax.experimental import pallas as pl
from jax.experimental.pallas import tpu as pltpu
from jax.experimental.pallas import tpu_sc as plsc


def _proj_body(rel_ref, w_ref, v_ref):
    s = jnp.sum(rel_ref[...] * w_ref[...], axis=1)
    v_ref[...] = jnp.concatenate([s, jnp.zeros((1,), jnp.float32)])


def _project(rel_pos, w_proj):
    n = rel_pos.shape[0]
    return pl.pallas_call(
        _proj_body,
        out_shape=jax.ShapeDtypeStruct((n + 1,), jnp.float32),
    )(rel_pos, w_proj)


def _make_expand(L, NC, NS):
    NW = NC * NS
    n_groups = L // 8
    assert n_groups % NW == 0
    g_per_w = n_groups // NW
    n_pad = 2 * L
    mesh = plsc.VectorSubcoreMesh(core_axis_name="c", subcore_axis_name="s")

    @functools.partial(
        pl.kernel,
        mesh=mesh,
        out_type=jax.ShapeDtypeStruct((L, L), jnp.float32),
        scratch_types=[
            pltpu.VMEM((n_pad + 16,), jnp.float32),
            pltpu.VMEM((8, L), jnp.float32),
            pltpu.VMEM((8, L), jnp.float32),
            pltpu.SemaphoreType.DMA,
            pltpu.SemaphoreType.DMA,
        ],
    )
    def expand(v_hbm, out_hbm, v_raw, buf0, buf1, sem0, sem1):
        wid = lax.axis_index("s") * NC + lax.axis_index("c")
        pltpu.sync_copy(v_hbm, v_raw.at[pl.ds(0, n_pad)])
        bufs = (buf0, buf1)
        sems = (sem0, sem1)

        copies = [None, None]
        for gg in range(g_per_w):
            G = wid * g_per_w + gg
            buf = bufs[gg % 2]
            if copies[gg % 2] is not None:
                copies[gg % 2].wait()

            base = (L - 8) - 8 * G

            def fill(t, _):
                off = base + 16 * t
                for i in range(8):
                    buf[i, pl.ds(16 * t, 16)] = v_raw[pl.ds(off + (7 - i), 16)]
                return 0

            lax.fori_loop(0, L // 16, fill, 0)

            row0 = pl.multiple_of(8 * G, 8)
            copies[gg % 2] = pltpu.async_copy(
                buf, out_hbm.at[pl.ds(row0, 8), :], sems[gg % 2]
            )
        for c in copies:
            c.wait()

    return expand


def kernel(rel_pos, W_proj, qlen, klen):
    L = (rel_pos.shape[0] + 1) // 2
    v = _project(rel_pos, W_proj)
    info = plsc.get_sparse_core_info()
    expand = _make_expand(L, info.num_cores, info.num_subcores)
    return expand(v)
